# Initial kernel scaffold; baseline (speedup 1.0000x reference)
import jax
import jax.numpy as jnp
from jax.experimental import pallas as pl


def kernel(x, edge_index, W1, a1_src, a1_dst, b1, W2, a2_src, a2_dst, b2, W3, a3_src, a3_dst, b3, Wf1, bf1, Wf2, bf2):
    raise NotImplementedError("write your pallas kernel here")


# speedup vs baseline: 187.4107x; 187.4107x over previous
import functools

import jax
import jax.numpy as jnp
from jax.experimental import pallas as pl

N = 512
E_TOTAL = 16384
HEADS = 4
HID = 32
HH = HEADS * HID
CHUNK = 4096

_DN_T = (((1,), (1,)), ((), ()))


def _leaky(x, slope):
    return jnp.where(x >= 0, x, slope * x)


def _gat_dense(xv, C, mask_neg, W, AselD, AselST, b):
    h = jnp.dot(xv, W, preferred_element_type=jnp.float32)
    ad = jnp.dot(h, AselD, preferred_element_type=jnp.float32)
    asT = jax.lax.dot_general(AselST, h, _DN_T,
                              preferred_element_type=jnp.float32)
    outs = []
    for hd in range(HEADS):
        ad_col = ad[:, hd:hd + 1]
        as_row = asT[hd:hd + 1, :]
        E = _leaky(ad_col + as_row, 0.2)
        m = jnp.max(E + mask_neg, axis=1, keepdims=True)
        P = C * jnp.exp(E - m)
        denom = jnp.sum(P, axis=1, keepdims=True)
        num = jnp.dot(P, h[:, hd * HID:(hd + 1) * HID],
                      preferred_element_type=jnp.float32)
        outs.append(num / denom)
    return jnp.concatenate(outs, axis=1) + b


def _body(src_row_ref, dst_row_ref, x_ref,
          W1_ref, S1_ref, D1_ref, b1_ref,
          W2_ref, S2_ref, D2_ref, b2_ref,
          W3_ref, S3_ref, D3_ref, b3_ref,
          Wf1a_ref, Wf1bT_ref, bf1_ref, Wf2_ref, bf2_ref,
          out_ref):
    iota_col = jax.lax.broadcasted_iota(jnp.int32, (N, 1), 0)
    C = jnp.zeros((N, N), dtype=jnp.float32)
    for c in range(E_TOTAL // CHUNK):
        src_chunk = src_row_ref[0:1, c * CHUNK:(c + 1) * CHUNK]
        dst_chunk = dst_row_ref[0:1, c * CHUNK:(c + 1) * CHUNK]
        src_ohT = (iota_col == src_chunk).astype(jnp.bfloat16)
        dst_ohT = (iota_col == dst_chunk).astype(jnp.bfloat16)
        C = C + jax.lax.dot_general(dst_ohT, src_ohT, _DN_T,
                                    preferred_element_type=jnp.float32)
    iota_row = jax.lax.broadcasted_iota(jnp.int32, (1, N), 1)
    C = C + (iota_col == iota_row).astype(jnp.float32)
    mask_neg = jnp.where(C > 0, 0.0, -1e30)

    x = x_ref[...]
    x1 = _leaky(_gat_dense(x, C, mask_neg, W1_ref[...], S1_ref[...],
                           D1_ref[...], b1_ref[...]), 0.01)
    x2 = _leaky(_gat_dense(x1, C, mask_neg, W2_ref[...], S2_ref[...],
                           D2_ref[...], b2_ref[...]), 0.01)
    x3 = _leaky(_gat_dense(x2, C, mask_neg, W3_ref[...], S3_ref[...],
                           D3_ref[...], b3_ref[...]), 0.01)

    A = jnp.dot(x3, Wf1a_ref[...], preferred_element_type=jnp.float32)
    BT = jax.lax.dot_general(Wf1bT_ref[...], x3, _DN_T,
                             preferred_element_type=jnp.float32)
    bf1 = bf1_ref[...]
    Wf2 = Wf2_ref[...]
    acc = jnp.zeros((N, N), dtype=jnp.float32) + bf2_ref[0, 0]
    for k in range(HID):
        t = jnp.maximum(A[:, k:k + 1] + BT[k:k + 1, :] + bf1[0:1, k:k + 1], 0.0)
        acc = acc + Wf2[0:1, k:k + 1] * t
    out_ref[...] = acc


@jax.jit
def kernel(x, edge_index, W1, a1_src, a1_dst, b1, W2, a2_src, a2_dst, b2,
           W3, a3_src, a3_dst, b3, Wf1, bf1, Wf2, bf2):
    ei = edge_index.astype(jnp.int32)
    src_row = ei[0].reshape(1, E_TOTAL)
    dst_row = ei[1].reshape(1, E_TOTAL)

    blk = (jnp.arange(HH, dtype=jnp.int32)[:, None] // HID
           == jnp.arange(HEADS, dtype=jnp.int32)[None, :]).astype(jnp.float32)

    def sel(a):
        return a.reshape(HH, 1) * blk

    args = (src_row, dst_row, x,
            W1, sel(a1_dst), sel(a1_src).T, b1.reshape(1, HH),
            W2, sel(a2_dst), sel(a2_src).T, b2.reshape(1, HH),
            W3, sel(a3_dst), sel(a3_src).T, b3.reshape(1, HH),
            Wf1[:HH], Wf1[HH:].T, bf1.reshape(1, HID),
            Wf2.reshape(1, HID), bf2.reshape(1, 1))

    return pl.pallas_call(
        _body,
        out_shape=jax.ShapeDtypeStruct((N, N), jnp.float32),
    )(*args)
